# baseline (speedup 1.0000x reference)
import jax
import jax.numpy as jnp
from jax import lax
from jax.experimental import pallas as pl
from jax.experimental.pallas import tpu as pltpu
from jax.experimental.pallas import tpu_sc as plsc

VOCAB = 100000
D = 128
B = 1024
S = 200
N = B * S

NC, NS = 2, 16
NW = NC * NS
PER_W = N // NW
NG = PER_W // D
NB = 6
LA = 4


def _gather_body(table_hbm, idx_hbm, out_hbm, idx_v, bufs, sem_in, sem_out):
    wid = lax.axis_index("s") * NC + lax.axis_index("c")
    pltpu.sync_copy(idx_hbm.at[wid], idx_v)
    base = wid * PER_W

    def issue_gather(i):
        pltpu.async_copy(table_hbm.at[idx_v.at[i]], bufs.at[i % NB], sem_in)

    def wait_in():
        pltpu.make_async_copy(
            table_hbm.at[pl.ds(0, D)], bufs.at[0], sem_in
        ).wait()

    def start_out(i):
        pltpu.async_copy(
            bufs.at[i % NB], out_hbm.at[pl.ds(base + i * D, D)], sem_out
        )

    def wait_out(i):
        pltpu.make_async_copy(
            bufs.at[i % NB], out_hbm.at[pl.ds(base + i * D, D)], sem_out
        ).wait()

    for i in range(LA):
        issue_gather(i)

    def head(i, carry):
        wait_in()
        start_out(i)
        issue_gather(i + LA)
        return carry

    lax.fori_loop(0, NB - LA, head, None)

    @pl.loop(NB - LA, NG - LA, unroll=2)
    def steady(i):
        wait_in()
        start_out(i)
        wait_out(i + LA - NB)
        issue_gather(i + LA)

    def tail(i, carry):
        wait_in()
        start_out(i)
        wait_out(i + LA - NB)
        return carry

    lax.fori_loop(NG - LA, NG, tail, None)
    for i in range(NG + LA - NB, NG):
        wait_out(i)


@jax.jit
def _gather(ids3d, table):
    run = pl.kernel(
        _gather_body,
        out_type=jax.ShapeDtypeStruct((N, D), jnp.float32),
        mesh=plsc.VectorSubcoreMesh(core_axis_name="c", subcore_axis_name="s"),
        scratch_types=[
            pltpu.VMEM((NG, D), jnp.int32),
            pltpu.VMEM((NB, D, D), jnp.float32),
            pltpu.SemaphoreType.DMA,
            pltpu.SemaphoreType.DMA,
        ],
    )
    return run(table, ids3d)


def kernel(input_ids, embedding):
    ids3d = input_ids.reshape(NW, NG, D).astype(jnp.int32)
    out = _gather(ids3d, embedding)
    return out.reshape(B, S, D)


# speedup vs baseline: 1.0039x; 1.0039x over previous
import jax
import jax.numpy as jnp
from jax import lax
from jax.experimental import pallas as pl
from jax.experimental.pallas import tpu as pltpu
from jax.experimental.pallas import tpu_sc as plsc

VOCAB = 100000
D = 128
B = 1024
S = 200
N = B * S

NC, NS = 2, 16
NW = NC * NS
PER_W = N // NW
NG = PER_W // D
NB = 6
LA = 4


def _gather_body(table_hbm, idx_hbm, out_hbm, idx_v, bufs, sem_in, sem_out):
    wid = lax.axis_index("s") * NC + lax.axis_index("c")
    pltpu.sync_copy(idx_hbm.at[wid], idx_v)
    base = wid * PER_W

    def issue_gather(i):
        pltpu.async_copy(table_hbm.at[idx_v.at[i]], bufs.at[i % NB], sem_in)

    def wait_in():
        pltpu.make_async_copy(
            table_hbm.at[pl.ds(0, D)], bufs.at[0], sem_in
        ).wait()

    def start_out(i):
        pltpu.async_copy(
            bufs.at[i % NB], out_hbm.at[pl.ds(base + i * D, D)], sem_out
        )

    def wait_out(i):
        pltpu.make_async_copy(
            bufs.at[i % NB], out_hbm.at[pl.ds(base + i * D, D)], sem_out
        ).wait()

    for i in range(LA):
        issue_gather(i)

    def head(i, carry):
        wait_in()
        start_out(i)
        issue_gather(i + LA)
        return carry

    lax.fori_loop(0, NB - LA, head, None)

    def steady(i, carry):
        wait_in()
        start_out(i)
        wait_out(i + LA - NB)
        issue_gather(i + LA)
        return carry

    lax.fori_loop(NB - LA, NG - LA, steady, None)

    def tail(i, carry):
        wait_in()
        start_out(i)
        wait_out(i + LA - NB)
        return carry

    lax.fori_loop(NG - LA, NG, tail, None)
    for i in range(NG + LA - NB, NG):
        wait_out(i)


@jax.jit
def _gather(ids3d, table):
    run = pl.kernel(
        _gather_body,
        out_type=jax.ShapeDtypeStruct((N, D), jnp.float32),
        mesh=plsc.VectorSubcoreMesh(core_axis_name="c", subcore_axis_name="s"),
        scratch_types=[
            pltpu.VMEM((NG, D), jnp.int32),
            pltpu.VMEM((NB, D, D), jnp.float32),
            pltpu.SemaphoreType.DMA,
            pltpu.SemaphoreType.DMA,
        ],
    )
    return run(table, ids3d)


def kernel(input_ids, embedding):
    ids3d = input_ids.reshape(NW, NG, D).astype(jnp.int32)
    out = _gather(ids3d, embedding)
    return out.reshape(B, S, D)
